# Initial kernel scaffold; baseline (speedup 1.0000x reference)
import jax
import jax.numpy as jnp
from jax.experimental import pallas as pl


def kernel(x, e):
    raise NotImplementedError("write your pallas kernel here")


# speedup vs baseline: 1.7931x; 1.7931x over previous
import jax
import jax.numpy as jnp
from jax.experimental import pallas as pl
from jax.experimental.pallas import tpu as pltpu
from jax.experimental.pallas import tpu_sc as plsc

_KT = 512


def _encode_body(x_ref, e_ref, codes_ref):
    K = e_ref.shape[0]
    N = x_ref.shape[2]
    xb = x_ref[0]
    x2 = jnp.sum(xb * xb, axis=0)

    def step(i, carry):
        best_val, best_idx = carry
        k0 = i * _KT
        ei = e_ref[pl.ds(k0, _KT), :]
        e2 = jnp.sum(ei * ei, axis=1, keepdims=True)
        cross = jax.lax.dot_general(
            ei.astype(jnp.bfloat16), xb.astype(jnp.bfloat16),
            (((1,), (0,)), ((), ())),
            preferred_element_type=jnp.float32)
        d2 = jnp.maximum(x2[None, :] + e2 - 2.0 * cross, 0.0)
        m = jnp.min(d2, axis=0)
        iota = jax.lax.broadcasted_iota(jnp.int32, (_KT, N), 0) + k0
        idx = jnp.min(jnp.where(d2 == m[None, :], iota, K), axis=0)
        better = m < best_val
        return (jnp.where(better, m, best_val),
                jnp.where(better, idx, best_idx))

    init = (jnp.full((N,), jnp.inf, jnp.float32),
            jnp.zeros((N,), jnp.int32))
    _, best_idx = jax.lax.fori_loop(0, K // _KT, step, init)
    codes_ref[0, 0, :] = best_idx


def _encode(x, e):
    B, D, N = x.shape
    K = e.shape[0]
    codes = pl.pallas_call(
        _encode_body,
        grid=(B,),
        in_specs=[pl.BlockSpec((1, D, N), lambda b: (b, 0, 0)),
                  pl.BlockSpec((K, D), lambda b: (0, 0))],
        out_specs=pl.BlockSpec((1, 1, N), lambda b: (b, 0, 0)),
        out_shape=jax.ShapeDtypeStruct((B, 1, N), jnp.int32),
    )(x, e)
    return codes.reshape(B * N)


def _decode_gather(e, codes):
    D = e.shape[1]
    n_tok = codes.shape[0]
    window = 128
    idx2 = codes.reshape(1, n_tok)
    mesh = plsc.VectorSubcoreMesh(core_axis_name="core",
                                  subcore_axis_name="subcore")

    @pl.kernel(out_type=jax.ShapeDtypeStruct((n_tok, D), e.dtype), mesh=mesh)
    def gather_kernel(e_hbm, i_hbm, o_hbm):
        def body(i_vmem, o_vmem):
            pltpu.sync_copy(e_hbm.at[i_vmem.at[0]], o_vmem)

        pltpu.emit_pipeline(
            body,
            grid=(n_tok // window,),
            in_specs=[pl.BlockSpec((1, window), index_map=lambda i: (0, i))],
            out_specs=[pl.BlockSpec((window, D), index_map=lambda i: (i, 0))],
            core_axis_name=("core", "subcore"),
            dimension_semantics=(pltpu.PARALLEL,),
        )(i_hbm, o_hbm)

    return gather_kernel(e, idx2)


def kernel(x, e):
    B, D, N = x.shape
    codes = _encode(x, e)
    q = _decode_gather(e, codes)
    return q.reshape(B, N, D).transpose(0, 2, 1)


# speedup vs baseline: 2.0207x; 1.1270x over previous
import jax
import jax.numpy as jnp
from jax.experimental import pallas as pl
from jax.experimental.pallas import tpu as pltpu
from jax.experimental.pallas import tpu_sc as plsc

_KT = 512


def _encode_body(x_ref, e_ref, codes_ref):
    K = e_ref.shape[0]
    N = x_ref.shape[2]
    xb = x_ref[0]
    x2 = jnp.sum(xb * xb, axis=0)
    xm2 = (-2.0 * xb).astype(jnp.bfloat16)
    iota_f = jax.lax.broadcasted_iota(jnp.int32, (_KT, N), 0).astype(jnp.float32)

    def step(i, carry):
        best_val, best_idx = carry
        k0 = i * _KT
        ei = e_ref[pl.ds(k0, _KT), :]
        e2 = jnp.sum(ei * ei, axis=1, keepdims=True)
        crossm2 = jax.lax.dot_general(
            ei.astype(jnp.bfloat16), xm2,
            (((1,), (0,)), ((), ())),
            preferred_element_type=jnp.float32)
        d2 = (x2[None, :] + e2) + crossm2
        m = jnp.min(d2, axis=0)
        idxf = jnp.min(
            jnp.where(d2 == m[None, :], iota_f, jnp.float32(K)), axis=0)
        better = m < best_val
        return (jnp.where(better, m, best_val),
                jnp.where(better, idxf + jnp.astype(k0, jnp.float32), best_idx))

    init = (jnp.full((N,), jnp.inf, jnp.float32),
            jnp.zeros((N,), jnp.float32))
    _, best_idx = jax.lax.fori_loop(0, K // _KT, step, init)
    codes_ref[0, 0, :] = best_idx.astype(jnp.int32)


def _encode(x, e):
    B, D, N = x.shape
    K = e.shape[0]
    codes = pl.pallas_call(
        _encode_body,
        grid=(B,),
        in_specs=[pl.BlockSpec((1, D, N), lambda b: (b, 0, 0)),
                  pl.BlockSpec((K, D), lambda b: (0, 0))],
        out_specs=pl.BlockSpec((1, 1, N), lambda b: (b, 0, 0)),
        out_shape=jax.ShapeDtypeStruct((B, 1, N), jnp.int32),
    )(x, e)
    return codes.reshape(B * N)


def _decode_gather(e, codes):
    D = e.shape[1]
    n_tok = codes.shape[0]
    window = 128
    idx2 = codes.reshape(1, n_tok)
    mesh = plsc.VectorSubcoreMesh(core_axis_name="core",
                                  subcore_axis_name="subcore")

    @pl.kernel(out_type=jax.ShapeDtypeStruct((n_tok, D), e.dtype), mesh=mesh)
    def gather_kernel(e_hbm, i_hbm, o_hbm):
        def body(i_vmem, o_vmem):
            pltpu.sync_copy(e_hbm.at[i_vmem.at[0]], o_vmem)

        pltpu.emit_pipeline(
            body,
            grid=(n_tok // window,),
            in_specs=[pl.BlockSpec((1, window), index_map=lambda i: (0, i))],
            out_specs=[pl.BlockSpec((window, D), index_map=lambda i: (i, 0))],
            core_axis_name=("core", "subcore"),
            dimension_semantics=(pltpu.PARALLEL,),
        )(i_hbm, o_hbm)

    return gather_kernel(e, idx2)


def kernel(x, e):
    B, D, N = x.shape
    codes = _encode(x, e)
    q = _decode_gather(e, codes)
    return q.reshape(B, N, D).transpose(0, 2, 1)
